# baseline (speedup 1.0000x reference)
import functools

import jax
import jax.numpy as jnp
from jax import lax
from jax.experimental import pallas as pl
from jax.experimental.pallas import tpu as pltpu
from jax.experimental.pallas import tpu_sc as plsc

B = 128
L = 8192
V = 100000
CW = 3200
NFULL = 31
NBUF = 4
NCORES = 2
SC_COLS = 99968
WMAX = 99840
TAILW = 2048
TAIL_I = 48

_CHUNKS = [(k * CW, CW) for k in range(NFULL)] + [(NFULL * CW, 768)]


def _gather_body(cnt_ref, sid_ref, tok_ref, newcnt_ref):
    cnt = cnt_ref[:, :]
    col = lax.broadcasted_iota(jnp.int32, (B, L), 1)
    hit = col == cnt
    tok_ref[:, :] = jnp.sum(jnp.where(hit, sid_ref[:, :], 0), axis=1, keepdims=True)
    newcnt_ref[:, :] = cnt + 1


def _sc_body(rp, out, b0, b1, b2, b3,
             s_in0, s_in1, s_in2, s_in3, s_out0, s_out1, s_out2, s_out3):
    bufs = (b0, b1, b2, b3)
    in_sems = (s_in0, s_in1, s_in2, s_in3)
    out_sems = (s_out0, s_out1, s_out2, s_out3)
    cid = lax.axis_index("c")
    sid_ = lax.axis_index("s")
    wid = sid_ * NCORES + cid
    slab = wid // 2
    half = wid % 2
    r0 = slab * 8

    def run_ring(my_chunks):
        def in_dma(j):
            c0, cw = my_chunks[j]
            s = j % NBUF
            return pltpu.make_async_copy(
                rp.at[pl.ds(r0, 8), pl.ds(c0, cw)],
                bufs[s].at[:, pl.ds(0, cw)],
                in_sems[s])

        def out_dma(j):
            c0, cw = my_chunks[j]
            s = j % NBUF
            return pltpu.make_async_copy(
                bufs[s].at[:, pl.ds(0, cw)],
                out.at[pl.ds(r0, 8), pl.ds(c0, cw)],
                out_sems[s])

        K = NBUF - 2
        NJ = len(my_chunks)
        for j in range(min(K, NJ)):
            in_dma(j).start()
        for j in range(NJ):
            in_dma(j).wait()
            out_dma(j).start()
            jn = j + K
            if jn < NJ:
                if jn - NBUF >= 0:
                    out_dma(jn - NBUF).wait()
                in_dma(jn).start()
        for j in range(max(NJ - NBUF, 0), NJ):
            out_dma(j).wait()

    @pl.when(half == 0)
    def _():
        run_ring(_CHUNKS[:16])

    @pl.when(half == 1)
    def _():
        run_ring(_CHUNKS[16:])


def _make_sc_kernel():
    mesh = plsc.VectorSubcoreMesh(core_axis_name="c", subcore_axis_name="s")
    return functools.partial(
        pl.kernel,
        mesh=mesh,
        out_type=jax.ShapeDtypeStruct((B, V), jnp.float32),
        scratch_types=[
            pltpu.VMEM((8, CW), jnp.float32),
            pltpu.VMEM((8, CW), jnp.float32),
            pltpu.VMEM((8, CW), jnp.float32),
            pltpu.VMEM((8, CW), jnp.float32),
            pltpu.SemaphoreType.DMA,
            pltpu.SemaphoreType.DMA,
            pltpu.SemaphoreType.DMA,
            pltpu.SemaphoreType.DMA,
            pltpu.SemaphoreType.DMA,
            pltpu.SemaphoreType.DMA,
            pltpu.SemaphoreType.DMA,
            pltpu.SemaphoreType.DMA,
        ],
    )(_sc_body)


def _patch_body(tok_ref, rp_ref, x_ref, o_ref, ibuf, in_sems, out_sems):
    del x_ref

    def wtile(b):
        t = tok_ref[b, 0]
        return pl.multiple_of(jnp.minimum((t >> 7) << 7, WMAX), 128)

    for b in range(B):
        g = b // 8
        w = wtile(b)
        pltpu.make_async_copy(
            rp_ref.at[pl.ds(8 * g, 8), pl.ds(w, 128)],
            ibuf.at[b], in_sems.at[b]).start()

    rowi = lax.broadcasted_iota(jnp.int32, (8, 128), 0)
    coli = lax.broadcasted_iota(jnp.int32, (8, 128), 1)
    for b in range(B):
        g = b // 8
        w = wtile(b)
        pltpu.make_async_copy(
            rp_ref.at[pl.ds(8 * g, 8), pl.ds(w, 128)],
            ibuf.at[b], in_sems.at[b]).wait()
        colg = coli + w
        hit = jnp.zeros((8, 128), jnp.bool_)
        for r in range(8):
            hit = jnp.logical_or(
                hit,
                jnp.logical_and(rowi == r, colg == tok_ref[8 * g + r, 0]))
        ibuf[b, :, :] = jnp.where(hit, jnp.float32(1.0), ibuf[b, :, :])
        pltpu.make_async_copy(
            ibuf.at[b], o_ref.at[pl.ds(8 * g, 8), pl.ds(w, 128)],
            out_sems.at[b]).start()

    for b in range(B):
        g = b // 8
        w = wtile(b)
        pltpu.make_async_copy(
            ibuf.at[b], o_ref.at[pl.ds(8 * g, 8), pl.ds(w, 128)],
            out_sems.at[b]).wait()


def _tail_body(tok_ref, rp_ref, x_ref, o_ref):
    del x_ref
    col = lax.broadcasted_iota(jnp.int32, (B, TAILW), 1) + TAIL_I * TAILW
    hit = col == tok_ref[:, :]
    o_ref[:, :] = jnp.where(hit, jnp.float32(1.0), rp_ref[:, :])


@jax.jit
def kernel(save_id, repeat_penality, penality_reset_count):
    tok, new_count = pl.pallas_call(
        _gather_body,
        out_shape=(
            jax.ShapeDtypeStruct((B, 1), save_id.dtype),
            jax.ShapeDtypeStruct((B, 1), penality_reset_count.dtype),
        ),
    )(penality_reset_count, save_id)

    sc_out = _make_sc_kernel()(repeat_penality)

    patched = pl.pallas_call(
        _patch_body,
        in_specs=[
            pl.BlockSpec(memory_space=pltpu.SMEM),
            pl.BlockSpec(memory_space=pl.ANY),
            pl.BlockSpec(memory_space=pl.ANY),
        ],
        out_specs=pl.BlockSpec(memory_space=pl.ANY),
        out_shape=jax.ShapeDtypeStruct((B, V), jnp.float32),
        input_output_aliases={2: 0},
        scratch_shapes=[
            pltpu.VMEM((B, 8, 128), jnp.float32),
            pltpu.SemaphoreType.DMA((B,)),
            pltpu.SemaphoreType.DMA((B,)),
        ],
    )(tok, repeat_penality, sc_out)

    out = pl.pallas_call(
        _tail_body,
        grid=(1,),
        in_specs=[
            pl.BlockSpec((B, 1), lambda i: (0, 0)),
            pl.BlockSpec((B, TAILW), lambda i: (0, TAIL_I)),
            pl.BlockSpec((B, TAILW), lambda i: (0, TAIL_I)),
        ],
        out_specs=pl.BlockSpec((B, TAILW), lambda i: (0, TAIL_I)),
        out_shape=jax.ShapeDtypeStruct((B, V), jnp.float32),
        input_output_aliases={2: 0},
    )(tok, repeat_penality, patched)
    return (out, new_count)


# speedup vs baseline: 1.0033x; 1.0033x over previous
import functools

import jax
import jax.numpy as jnp
from jax import lax
from jax.experimental import pallas as pl
from jax.experimental.pallas import tpu as pltpu
from jax.experimental.pallas import tpu_sc as plsc

B = 128
L = 8192
V = 100000
CW = 3200
NFULL = 31
NBUF = 4
NCORES = 2
SC_COLS = 99968
WMAX = 99840
TAILW = 2048
TAIL_I = 48

_CHUNKS = [(k * CW, CW) for k in range(NFULL)] + [(NFULL * CW, 768)]


def _gather_body(cnt_ref, sid_ref, tok_ref, newcnt_ref):
    cnt = cnt_ref[:, :]
    col = lax.broadcasted_iota(jnp.int32, (B, L), 1)
    hit = col == cnt
    tok_ref[:, :] = jnp.sum(jnp.where(hit, sid_ref[:, :], 0), axis=1, keepdims=True)
    newcnt_ref[:, :] = cnt + 1


def _sc_body(rp, out, b0, b1, b2, b3,
             s_in0, s_in1, s_in2, s_in3, s_out0, s_out1, s_out2, s_out3):
    bufs = (b0, b1, b2, b3)
    in_sems = (s_in0, s_in1, s_in2, s_in3)
    out_sems = (s_out0, s_out1, s_out2, s_out3)
    cid = lax.axis_index("c")
    sid_ = lax.axis_index("s")
    wid = sid_ * NCORES + cid
    slab = wid // 2
    half = wid % 2
    r0 = slab * 8

    def run_ring(my_chunks):
        def in_dma(j):
            c0, cw = my_chunks[j]
            s = j % NBUF
            return pltpu.make_async_copy(
                rp.at[pl.ds(r0, 8), pl.ds(c0, cw)],
                bufs[s].at[:, pl.ds(0, cw)],
                in_sems[s])

        def out_dma(j):
            c0, cw = my_chunks[j]
            s = j % NBUF
            return pltpu.make_async_copy(
                bufs[s].at[:, pl.ds(0, cw)],
                out.at[pl.ds(r0, 8), pl.ds(c0, cw)],
                out_sems[s])

        K = NBUF - 1
        NJ = len(my_chunks)
        for j in range(min(K, NJ)):
            in_dma(j).start()
        for j in range(NJ):
            in_dma(j).wait()
            out_dma(j).start()
            jn = j + K
            if jn < NJ:
                if jn - NBUF >= 0:
                    out_dma(jn - NBUF).wait()
                in_dma(jn).start()
        for j in range(max(NJ - NBUF, 0), NJ):
            out_dma(j).wait()

    @pl.when(half == 0)
    def _():
        run_ring(_CHUNKS[:16])

    @pl.when(half == 1)
    def _():
        run_ring(_CHUNKS[16:])


def _make_sc_kernel():
    mesh = plsc.VectorSubcoreMesh(core_axis_name="c", subcore_axis_name="s")
    return functools.partial(
        pl.kernel,
        mesh=mesh,
        out_type=jax.ShapeDtypeStruct((B, V), jnp.float32),
        scratch_types=[
            pltpu.VMEM((8, CW), jnp.float32),
            pltpu.VMEM((8, CW), jnp.float32),
            pltpu.VMEM((8, CW), jnp.float32),
            pltpu.VMEM((8, CW), jnp.float32),
            pltpu.SemaphoreType.DMA,
            pltpu.SemaphoreType.DMA,
            pltpu.SemaphoreType.DMA,
            pltpu.SemaphoreType.DMA,
            pltpu.SemaphoreType.DMA,
            pltpu.SemaphoreType.DMA,
            pltpu.SemaphoreType.DMA,
            pltpu.SemaphoreType.DMA,
        ],
    )(_sc_body)


def _patch_body(tok_ref, rp_ref, x_ref, o_ref, ibuf, in_sems, out_sems):
    del x_ref

    def wtile(b):
        t = tok_ref[b, 0]
        return pl.multiple_of(jnp.minimum((t >> 7) << 7, WMAX), 128)

    for b in range(B):
        g = b // 8
        w = wtile(b)
        pltpu.make_async_copy(
            rp_ref.at[pl.ds(8 * g, 8), pl.ds(w, 128)],
            ibuf.at[b], in_sems.at[b]).start()

    rowi = lax.broadcasted_iota(jnp.int32, (8, 128), 0)
    coli = lax.broadcasted_iota(jnp.int32, (8, 128), 1)
    for b in range(B):
        g = b // 8
        w = wtile(b)
        pltpu.make_async_copy(
            rp_ref.at[pl.ds(8 * g, 8), pl.ds(w, 128)],
            ibuf.at[b], in_sems.at[b]).wait()
        colg = coli + w
        hit = jnp.zeros((8, 128), jnp.bool_)
        for r in range(8):
            hit = jnp.logical_or(
                hit,
                jnp.logical_and(rowi == r, colg == tok_ref[8 * g + r, 0]))
        ibuf[b, :, :] = jnp.where(hit, jnp.float32(1.0), ibuf[b, :, :])
        pltpu.make_async_copy(
            ibuf.at[b], o_ref.at[pl.ds(8 * g, 8), pl.ds(w, 128)],
            out_sems.at[b]).start()

    for b in range(B):
        g = b // 8
        w = wtile(b)
        pltpu.make_async_copy(
            ibuf.at[b], o_ref.at[pl.ds(8 * g, 8), pl.ds(w, 128)],
            out_sems.at[b]).wait()


def _tail_body(tok_ref, rp_ref, x_ref, o_ref):
    del x_ref
    col = lax.broadcasted_iota(jnp.int32, (B, TAILW), 1) + TAIL_I * TAILW
    hit = col == tok_ref[:, :]
    o_ref[:, :] = jnp.where(hit, jnp.float32(1.0), rp_ref[:, :])


@jax.jit
def kernel(save_id, repeat_penality, penality_reset_count):
    tok, new_count = pl.pallas_call(
        _gather_body,
        out_shape=(
            jax.ShapeDtypeStruct((B, 1), save_id.dtype),
            jax.ShapeDtypeStruct((B, 1), penality_reset_count.dtype),
        ),
    )(penality_reset_count, save_id)

    sc_out = _make_sc_kernel()(repeat_penality)

    patched = pl.pallas_call(
        _patch_body,
        in_specs=[
            pl.BlockSpec(memory_space=pltpu.SMEM),
            pl.BlockSpec(memory_space=pl.ANY),
            pl.BlockSpec(memory_space=pl.ANY),
        ],
        out_specs=pl.BlockSpec(memory_space=pl.ANY),
        out_shape=jax.ShapeDtypeStruct((B, V), jnp.float32),
        input_output_aliases={2: 0},
        scratch_shapes=[
            pltpu.VMEM((B, 8, 128), jnp.float32),
            pltpu.SemaphoreType.DMA((B,)),
            pltpu.SemaphoreType.DMA((B,)),
        ],
    )(tok, repeat_penality, sc_out)

    out = pl.pallas_call(
        _tail_body,
        grid=(1,),
        in_specs=[
            pl.BlockSpec((B, 1), lambda i: (0, 0)),
            pl.BlockSpec((B, TAILW), lambda i: (0, TAIL_I)),
            pl.BlockSpec((B, TAILW), lambda i: (0, TAIL_I)),
        ],
        out_specs=pl.BlockSpec((B, TAILW), lambda i: (0, TAIL_I)),
        out_shape=jax.ShapeDtypeStruct((B, V), jnp.float32),
        input_output_aliases={2: 0},
    )(tok, repeat_penality, patched)
    return (out, new_count)
